# Initial kernel scaffold; baseline (speedup 1.0000x reference)
import jax
import jax.numpy as jnp
from jax.experimental import pallas as pl


def kernel(x, edge_index, batch, emb, W_rel1, b_rel1, W_root1, W_rel2, b_rel2, W_root2, W_rel3, b_rel3, W_root3, W_rel4, b_rel4, W_root4, W_lin, b_lin):
    raise NotImplementedError("write your pallas kernel here")


# speedup vs baseline: 7.6729x; 7.6729x over previous
import functools

import jax
import jax.numpy as jnp
from jax import lax
from jax.experimental import pallas as pl
from jax.experimental.pallas import tpu as pltpu
from jax.experimental.pallas import tpu_sc as plsc

N = 50000
E = 800000
G = 64
NLAB = 10

NC = 2
NS = 16
NW = NC * NS
NPAD = 50176
RPT = NPAD // NS
CB = 128
ECH = 196
EPW = ECH * CB
EPAD = EPW * NW

BM = 1024
GRID = NPAD // BM

_f32 = jnp.float32


def _leaky(z):
    return jnp.where(z >= 0, z, 0.01 * z)


def _make_segsum(d):
    mesh = plsc.VectorSubcoreMesh(core_axis_name="c", subcore_axis_name="s")

    @functools.partial(
        pl.kernel,
        out_type=jax.ShapeDtypeStruct((NC, NPAD, d), _f32),
        mesh=mesh,
        scratch_types=[
            pltpu.VMEM((ECH, CB), jnp.int32),
            pltpu.VMEM((ECH, CB), jnp.int32),
            pltpu.VMEM((CB, d), _f32),
            pltpu.VMEM((CB, d), _f32),
            pltpu.VMEM_SHARED((NPAD, d), _f32),
            pltpu.SemaphoreType.DMA,
            pltpu.SemaphoreType.DMA,
        ],
        compiler_params=pltpu.CompilerParams(use_tc_tiling_on_sc=False),
    )
    def segsum(h_hbm, src_hbm, dst_hbm, zeros_hbm, out_hbm,
               src_v, dst_v, rows_a, rows_b, agg_sh, sem_a, sem_b):
        cid = lax.axis_index("c")
        sid = lax.axis_index("s")
        wid = sid * NC + cid
        pltpu.sync_copy(zeros_hbm, agg_sh.at[pl.ds(sid * RPT, RPT)])
        pltpu.sync_copy(src_hbm.at[wid], src_v)
        pltpu.sync_copy(dst_hbm.at[wid], dst_v)
        plsc.subcore_barrier()

        pltpu.async_copy(h_hbm.at[src_v.at[0]], rows_a, sem_a)

        def body(k, carry):
            j = 2 * k
            pltpu.async_copy(h_hbm.at[src_v.at[j + 1]], rows_b, sem_b)
            pltpu.make_async_copy(h_hbm.at[src_v.at[j]], rows_a, sem_a).wait()
            pltpu.sync_copy(rows_a, agg_sh.at[dst_v.at[j]], add=True)

            @pl.when(j + 2 < ECH)
            def _():
                pltpu.async_copy(h_hbm.at[src_v.at[j + 2]], rows_a, sem_a)

            pltpu.make_async_copy(h_hbm.at[src_v.at[j + 1]], rows_b, sem_b).wait()
            pltpu.sync_copy(rows_b, agg_sh.at[dst_v.at[j + 1]], add=True)
            return carry

        lax.fori_loop(0, ECH // 2, body, 0)

        plsc.subcore_barrier()
        pltpu.sync_copy(agg_sh.at[pl.ds(sid * RPT, RPT)],
                        out_hbm.at[cid, pl.ds(sid * RPT, RPT)])

    return segsum


def _dot(a, b, prec=lax.Precision.DEFAULT):
    return lax.dot_general(a, b, (((a.ndim - 1,), (0,)), ((), ())),
                           precision=prec, preferred_element_type=_f32)


def _embed_body(x_ref, emb_ref, out_ref):
    lab = x_ref[...]
    oh = (lab == lax.broadcasted_iota(jnp.int32, (BM, NLAB), 1)).astype(_f32)
    out_ref[...] = _dot(oh, emb_ref[...], lax.Precision.HIGHEST)


def _layer_body(a0_ref, a1_ref, h_ref, wr_ref, b_ref, wo_ref, out_ref):
    agg = a0_ref[...] + a1_ref[...]
    z = _dot(agg, wr_ref[...]) + b_ref[...] + _dot(h_ref[...], wo_ref[...])
    out_ref[...] = _leaky(z)


def _layer3_body(a00, a01, a10, a11, a20, a21, a30, a31, h_ref, wr_ref, b_ref,
                 wo_ref, bt_ref, wl_ref, bl_ref,
                 lat_ref, logits_ref, acc_s, acc_c):
    i = pl.program_id(0)
    agg = jnp.concatenate([a00[...] + a01[...], a10[...] + a11[...],
                           a20[...] + a21[...], a30[...] + a31[...]],
                          axis=1)
    z = _dot(agg, wr_ref[...]) + b_ref[...] + _dot(h_ref[...], wo_ref[...])
    lat = _leaky(z)
    lat_ref[...] = lat

    bt = bt_ref[...]
    oh = (bt == lax.broadcasted_iota(jnp.int32, (BM, G), 1)).astype(_f32)
    s = lax.dot_general(oh, lat, (((0,), (0,)), ((), ())),
                        precision=lax.Precision.HIGHEST,
                        preferred_element_type=_f32)
    c = lax.dot_general(oh, jnp.ones((BM, 1), _f32), (((0,), (0,)), ((), ())),
                        precision=lax.Precision.HIGHEST,
                        preferred_element_type=_f32)

    @pl.when(i == 0)
    def _():
        acc_s[...] = jnp.zeros_like(acc_s)
        acc_c[...] = jnp.zeros_like(acc_c)

    acc_s[...] += s
    acc_c[...] += c

    @pl.when(i == GRID - 1)
    def _():
        pooled = acc_s[...] / jnp.maximum(acc_c[...], 1.0)
        logits_ref[...] = _dot(pooled, wl_ref[...]) + bl_ref[...]


def _row_spec(d):
    return pl.BlockSpec((BM, d), lambda i: (i, 0))


def _full_spec(shape):
    return pl.BlockSpec(shape, lambda i: (0,) * len(shape))


def kernel(x, edge_index, batch, emb,
           W_rel1, b_rel1, W_root1,
           W_rel2, b_rel2, W_root2,
           W_rel3, b_rel3, W_root3,
           W_rel4, b_rel4, W_root4,
           W_lin, b_lin):
    xp = jnp.pad(x.astype(jnp.int32), ((0, NPAD - N), (0, 0)))
    npad_e = EPAD - E
    pad_idx = N + (jnp.arange(npad_e, dtype=jnp.int32) % (NPAD - N))
    src = jnp.concatenate([edge_index[0], pad_idx])
    dst = jnp.concatenate([edge_index[1], pad_idx])
    srcA = src.reshape(NW, ECH, CB)
    dstA = dst.reshape(NW, ECH, CB)
    batchp = jnp.pad(batch, (0, NPAD - N), constant_values=G).reshape(NPAD, 1)
    z16 = jnp.zeros((RPT, 16), _f32)

    embp = jnp.pad(emb, ((0, 0), (0, 13)))
    wr1 = jnp.pad(W_rel1, ((0, 12), (0, 13))).T
    wo1 = jnp.pad(W_root1, ((0, 12), (0, 13))).T
    wr2 = jnp.pad(W_rel2, ((0, 0), (0, 12))).T
    wo2 = jnp.pad(W_root2, ((0, 0), (0, 12))).T
    wr3, wo3 = W_rel3.T, W_root3.T
    wr4, wo4 = W_rel4.T, W_root4.T
    wl = W_lin.T
    b1 = jnp.pad(b_rel1, (0, 12)).reshape(1, -1)
    b2 = b_rel2.reshape(1, -1)
    b3 = b_rel3.reshape(1, -1)
    b4 = b_rel4.reshape(1, -1)
    bl = b_lin.reshape(1, -1)

    segsum16 = _make_segsum(16)

    h0 = pl.pallas_call(
        _embed_body,
        grid=(GRID,),
        in_specs=[_row_spec(1), _full_spec((NLAB, 16))],
        out_specs=_row_spec(16),
        out_shape=jax.ShapeDtypeStruct((NPAD, 16), _f32),
    )(xp, embp)

    def dense_layer(a, h, wr, b, wo, dout):
        return pl.pallas_call(
            _layer_body,
            grid=(GRID,),
            in_specs=[_row_spec(a.shape[-1]), _row_spec(a.shape[-1]),
                      _row_spec(h.shape[-1]),
                      _full_spec(wr.shape), _full_spec(b.shape),
                      _full_spec(wo.shape)],
            out_specs=_row_spec(dout),
            out_shape=jax.ShapeDtypeStruct((NPAD, dout), _f32),
        )(a[0], a[1], h, wr, b, wo)

    agg1 = segsum16(h0, srcA, dstA, z16)
    h1 = dense_layer(agg1, h0, wr1, b1, wo1, 16)

    agg2 = segsum16(h1, srcA, dstA, z16)
    orig = dense_layer(agg2, h1, wr2, b2, wo2, 64)

    agg3 = [segsum16(orig[:, 16 * q:16 * (q + 1)], srcA, dstA, z16)
            for q in range(4)]
    latent, logits = pl.pallas_call(
        _layer3_body,
        grid=(GRID,),
        in_specs=[_row_spec(16)] * 8 + [_row_spec(64),
                  _full_spec(wr3.shape), _full_spec(b3.shape),
                  _full_spec(wo3.shape),
                  _row_spec(1),
                  _full_spec(wl.shape), _full_spec(bl.shape)],
        out_specs=[_row_spec(16), _full_spec((G, 7))],
        out_shape=[jax.ShapeDtypeStruct((NPAD, 16), _f32),
                   jax.ShapeDtypeStruct((G, 7), _f32)],
        scratch_shapes=[pltpu.VMEM((G, 16), _f32), pltpu.VMEM((G, 1), _f32)],
    )(agg3[0][0], agg3[0][1], agg3[1][0], agg3[1][1],
      agg3[2][0], agg3[2][1], agg3[3][0], agg3[3][1],
      orig, wr3, b3, wo3, batchp, wl, bl)

    agg4 = segsum16(latent, srcA, dstA, z16)
    recon = dense_layer(agg4, latent, wr4, b4, wo4, 64)

    return (logits, recon[:N], orig[:N])
